# Initial kernel scaffold; baseline (speedup 1.0000x reference)
import jax
import jax.numpy as jnp
from jax.experimental import pallas as pl


def kernel(nodes, edges, edge_attr, g0_ew1, g0_eb1, g0_ew2, g0_eb2, g0_nw1, g0_nb1, g0_nw2, g0_nb2, g1_ew1, g1_eb1, g1_ew2, g1_eb2, g1_nw1, g1_nb1, g1_nw2, g1_nb2, emb_w, emb_b):
    raise NotImplementedError("write your pallas kernel here")


# speedup vs baseline: 3.4536x; 3.4536x over previous
import functools

import jax
import jax.numpy as jnp
from jax import lax
from jax.experimental import pallas as pl
from jax.experimental.pallas import tpu as pltpu
from jax.experimental.pallas import tpu_sc as plsc

N = 4096
E = 65536
H = 256
EMB = 8

NC = 2
NS = 16
NW = NC * NS
PW = E // NW
CH = 128
NCHUNK = PW // CH
RPT = N // NS

_f32 = jnp.float32
_i32 = jnp.int32


def _mesh():
    return plsc.VectorSubcoreMesh(core_axis_name="c", subcore_axis_name="s")


def _sc_gather_scalar(nodes1d, rows, cols):
    @functools.partial(
        pl.kernel, mesh=_mesh(),
        out_type=(jax.ShapeDtypeStruct((E,), _f32),
                  jax.ShapeDtypeStruct((E,), _f32)),
        scratch_types=[
            pltpu.VMEM((CH,), _i32), pltpu.VMEM((CH,), _i32),
            pltpu.VMEM((PW,), _f32), pltpu.VMEM((PW,), _f32),
            pltpu.SemaphoreType.DMA, pltpu.SemaphoreType.DMA,
        ],
    )
    def k(nodes_hbm, rows_hbm, cols_hbm, gr_hbm, gc_hbm, ri, ci, gr, gc, sa, sb):
        wid = lax.axis_index("s") * NC + lax.axis_index("c")
        base = wid * PW

        def chunk(kk, carry):
            o = kk * CH
            pltpu.sync_copy(rows_hbm.at[pl.ds(base + o, CH)], ri)
            pltpu.sync_copy(cols_hbm.at[pl.ds(base + o, CH)], ci)
            cpa = pltpu.async_copy(nodes_hbm.at[ri], gr.at[pl.ds(o, CH)], sa)
            cpb = pltpu.async_copy(nodes_hbm.at[ci], gc.at[pl.ds(o, CH)], sb)
            cpa.wait()
            cpb.wait()
            return carry

        lax.fori_loop(0, NCHUNK, chunk, 0)
        pltpu.sync_copy(gr, gr_hbm.at[pl.ds(base, PW)])
        pltpu.sync_copy(gc, gc_hbm.at[pl.ds(base, PW)])

    return k(nodes1d, rows, cols)


def _sc_gather_pairs(A, B, rows, cols):
    @functools.partial(
        pl.kernel, mesh=_mesh(),
        out_type=jax.ShapeDtypeStruct((E, H), _f32),
        scratch_types=[
            pltpu.VMEM((CH,), _i32), pltpu.VMEM((CH,), _i32),
            pltpu.VMEM((CH, H), _f32), pltpu.VMEM((CH, H), _f32),
            pltpu.SemaphoreType.DMA, pltpu.SemaphoreType.DMA,
        ],
    )
    def k(a_hbm, b_hbm, rows_hbm, cols_hbm, g_hbm, ri, ci, av, bv, sa, sb):
        wid = lax.axis_index("s") * NC + lax.axis_index("c")
        base = wid * PW

        def chunk(kk, carry):
            cb = base + kk * CH
            pltpu.sync_copy(rows_hbm.at[pl.ds(cb, CH)], ri)
            pltpu.sync_copy(cols_hbm.at[pl.ds(cb, CH)], ci)
            cpa = pltpu.async_copy(a_hbm.at[ri], av, sa)
            cpb = pltpu.async_copy(b_hbm.at[ci], bv, sb)
            cpa.wait()
            cpb.wait()

            def addrow(t, c2):
                e = t // (H // 16)
                j = t % (H // 16)
                av[e, pl.ds(j * 16, 16)] = (av[e, pl.ds(j * 16, 16)]
                                            + bv[e, pl.ds(j * 16, 16)])
                return c2

            lax.fori_loop(0, CH * (H // 16), addrow, 0)
            pltpu.sync_copy(av, g_hbm.at[pl.ds(cb, CH)])
            return carry

        lax.fori_loop(0, NCHUNK, chunk, 0)

    return k(A, B, rows, cols)


HH = H // NC
PS = E // NS
NCHUNK2 = PS // CH


def _sc_scatter_add(m2, rows):
    @functools.partial(
        pl.kernel, mesh=_mesh(),
        out_type=jax.ShapeDtypeStruct((N, H), _f32),
        scratch_types=[
            pltpu.VMEM((CH,), _i32),
            pltpu.VMEM((CH, HH), _f32),
            pltpu.VMEM((CH, HH), _f32),
            pltpu.VMEM_SHARED((N, HH), _f32),
        ],
    )
    def k(m_hbm, rows_hbm, agg_hbm, ri, mv, zv, acc):
        cid = lax.axis_index("c")
        sid = lax.axis_index("s")
        base = sid * PS
        fcol = cid * HH

        def zbody(t, carry):
            e = t // (HH // 16)
            j = t % (HH // 16)
            zv[e, pl.ds(j * 16, 16)] = jnp.zeros((16,), _f32)
            return carry

        lax.fori_loop(0, CH * (HH // 16), zbody, 0)
        pltpu.sync_copy(zv, acc.at[pl.ds(sid * RPT, CH)])
        pltpu.sync_copy(zv, acc.at[pl.ds(sid * RPT + CH, CH)])
        plsc.subcore_barrier()

        def chunk(kk, carry):
            cb = base + kk * CH
            pltpu.sync_copy(rows_hbm.at[pl.ds(cb, CH)], ri)
            pltpu.sync_copy(m_hbm.at[pl.ds(cb, CH), pl.ds(fcol, HH)], mv)
            pltpu.sync_copy(mv, acc.at[ri], add=True)
            return carry

        lax.fori_loop(0, NCHUNK2, chunk, 0)
        plsc.subcore_barrier()
        pltpu.sync_copy(acc.at[pl.ds(sid * RPT, RPT)],
                        agg_hbm.at[pl.ds(sid * RPT, RPT), pl.ds(fcol, HH)])

    return k(m2, rows)


_BE = 2048


def _tc_edge0(gr, gc, ea, w_r, w_c, w_e, b1, w2, b2):
    def body(gr_ref, gc_ref, ea_ref, wr, wc, we, b1r, w2r, b2r, out):
        z = (gr_ref[...] * wr[...] + gc_ref[...] * wc[...]
             + ea_ref[...] * we[...] + b1r[...])
        m = jnp.maximum(z, 0.0)
        mm = jnp.dot(m, w2r[...], preferred_element_type=_f32) + b2r[...]
        out[...] = jnp.maximum(mm, 0.0)

    col = pl.BlockSpec((_BE, 1), lambda i: (i, 0))
    whole = lambda shape: pl.BlockSpec(shape, lambda i: (0,) * len(shape))
    return pl.pallas_call(
        body,
        grid=(E // _BE,),
        in_specs=[col, col, col, whole((1, H)), whole((1, H)), whole((1, H)),
                  whole((1, H)), whole((H, H)), whole((1, H))],
        out_specs=pl.BlockSpec((_BE, H), lambda i: (i, 0)),
        out_shape=jax.ShapeDtypeStruct((E, H), _f32),
    )(gr, gc, ea, w_r, w_c, w_e, b1, w2, b2)


def _tc_edge1(G, ea, w_e, w2, b2):
    def body(g_ref, ea_ref, we, w2r, b2r, out):
        z = g_ref[...] + ea_ref[...] * we[...]
        m = jnp.maximum(z, 0.0)
        mm = jnp.dot(m, w2r[...], preferred_element_type=_f32) + b2r[...]
        out[...] = jnp.maximum(mm, 0.0)

    whole = lambda shape: pl.BlockSpec(shape, lambda i: (0,) * len(shape))
    return pl.pallas_call(
        body,
        grid=(E // _BE,),
        in_specs=[pl.BlockSpec((_BE, H), lambda i: (i, 0)),
                  pl.BlockSpec((_BE, 1), lambda i: (i, 0)),
                  whole((1, H)), whole((H, H)), whole((1, H))],
        out_specs=pl.BlockSpec((_BE, H), lambda i: (i, 0)),
        out_shape=jax.ShapeDtypeStruct((E, H), _f32),
    )(G, ea, w_e, w2, b2)


_BN = 512


def _tc_node0(agg, nodes, w_h, w_a, b1, w2, b2, wA, bA, wB):
    def body(aggr, nd, whr, war, b1r, w2r, b2r, wAr, bAr, wBr, h1o, ao, bo):
        t = (nd[...] * whr[...]
             + jnp.dot(aggr[...], war[...], preferred_element_type=_f32) + b1r[...])
        u = jnp.maximum(t, 0.0)
        h = jnp.dot(u, w2r[...], preferred_element_type=_f32) + b2r[...] + nd[...]
        h1o[...] = h
        ao[...] = jnp.dot(h, wAr[...], preferred_element_type=_f32) + bAr[...]
        bo[...] = jnp.dot(h, wBr[...], preferred_element_type=_f32)

    whole = lambda shape: pl.BlockSpec(shape, lambda i: (0,) * len(shape))
    blk = pl.BlockSpec((_BN, H), lambda i: (i, 0))
    return pl.pallas_call(
        body,
        grid=(N // _BN,),
        in_specs=[blk,
                  pl.BlockSpec((_BN, 1), lambda i: (i, 0)),
                  whole((1, H)), whole((H, H)), whole((1, H)),
                  whole((H, H)), whole((1, H)),
                  whole((H, H)), whole((1, H)), whole((H, H))],
        out_specs=(blk, blk, blk),
        out_shape=(jax.ShapeDtypeStruct((N, H), _f32),
                   jax.ShapeDtypeStruct((N, H), _f32),
                   jax.ShapeDtypeStruct((N, H), _f32)),
    )(agg, nodes, w_h, w_a, b1, w2, b2, wA, bA, wB)


def _tc_node1(agg, h1, w_h, w_a, b1, w2, b2, ew, eb):
    def body(aggr, h1r, whr, war, b1r, w2r, b2r, ewr, ebr, xo):
        t = (jnp.dot(h1r[...], whr[...], preferred_element_type=_f32)
             + jnp.dot(aggr[...], war[...], preferred_element_type=_f32) + b1r[...])
        u = jnp.maximum(t, 0.0)
        h = (jnp.dot(u, w2r[...], preferred_element_type=_f32) + b2r[...]
             + h1r[...])
        xo[...] = jnp.dot(h, ewr[...], preferred_element_type=_f32) + ebr[...]

    whole = lambda shape: pl.BlockSpec(shape, lambda i: (0,) * len(shape))
    blk = pl.BlockSpec((_BN, H), lambda i: (i, 0))
    return pl.pallas_call(
        body,
        grid=(N // _BN,),
        in_specs=[blk, blk,
                  whole((H, H)), whole((H, H)), whole((1, H)),
                  whole((H, H)), whole((1, H)),
                  whole((H, EMB)), whole((1, EMB))],
        out_specs=pl.BlockSpec((_BN, EMB), lambda i: (i, 0)),
        out_shape=jax.ShapeDtypeStruct((N, EMB), _f32),
    )(agg, h1, w_h, w_a, b1, w2, b2, ew, eb)


_BR = 256


def _tc_decode(x, xT):
    def body(x_ref, xt_ref, out):
        xb = x_ref[...]
        xt = xt_ref[...]
        rb = jnp.sum(xb * xb, axis=1, keepdims=True)
        ra = jnp.sum(xt * xt, axis=0, keepdims=True)
        s = jnp.dot(xb, xt, preferred_element_type=_f32)
        t = 10.0 * (rb + ra - 2.0 * s) - 1.0
        sig = 1.0 / (1.0 + jnp.exp(-t))
        i = pl.program_id(0)
        rowid = lax.broadcasted_iota(_i32, (_BR, N), 0) + i * _BR
        colid = lax.broadcasted_iota(_i32, (_BR, N), 1)
        out[...] = jnp.where(rowid == colid, 0.0, sig)

    return pl.pallas_call(
        body,
        grid=(N // _BR,),
        in_specs=[pl.BlockSpec((_BR, EMB), lambda i: (i, 0)),
                  pl.BlockSpec((EMB, N), lambda i: (0, 0))],
        out_specs=pl.BlockSpec((_BR, N), lambda i: (i, 0)),
        out_shape=jax.ShapeDtypeStruct((N, N), _f32),
    )(x, xT)


def kernel(nodes, edges, edge_attr,
           g0_ew1, g0_eb1, g0_ew2, g0_eb2, g0_nw1, g0_nb1, g0_nw2, g0_nb2,
           g1_ew1, g1_eb1, g1_ew2, g1_eb2, g1_nw1, g1_nb1, g1_nw2, g1_nb2,
           emb_w, emb_b):
    rows = edges[0]
    cols = edges[1]

    gr, gc = _sc_gather_scalar(nodes.reshape(N), rows, cols)
    m2 = _tc_edge0(gr.reshape(E, 1), gc.reshape(E, 1), edge_attr,
                   g0_ew1[0:1], g0_ew1[1:2], g0_ew1[2:3],
                   g0_eb1.reshape(1, H), g0_ew2, g0_eb2.reshape(1, H))
    agg0 = _sc_scatter_add(m2, rows)
    h1, A1, B1 = _tc_node0(agg0, nodes,
                           g0_nw1[0:1], g0_nw1[1:], g0_nb1.reshape(1, H),
                           g0_nw2, g0_nb2.reshape(1, H),
                           g1_ew1[0:H], g1_eb1.reshape(1, H), g1_ew1[H:2 * H])

    G = _sc_gather_pairs(A1, B1, rows, cols)
    m2b = _tc_edge1(G, edge_attr, g1_ew1[2 * H:2 * H + 1],
                    g1_ew2, g1_eb2.reshape(1, H))
    agg1 = _sc_scatter_add(m2b, rows)
    x = _tc_node1(agg1, h1,
                  g1_nw1[0:H], g1_nw1[H:], g1_nb1.reshape(1, H),
                  g1_nw2, g1_nb2.reshape(1, H),
                  emb_w, emb_b.reshape(1, EMB))

    adj = _tc_decode(x, x.T)
    return adj, x


# speedup vs baseline: 5.5147x; 1.5968x over previous
import functools

import jax
import jax.numpy as jnp
from jax import lax
from jax.experimental import pallas as pl
from jax.experimental.pallas import tpu as pltpu
from jax.experimental.pallas import tpu_sc as plsc

N = 4096
E = 65536
H = 256
EMB = 8

NC = 2
NS = 16
NW = NC * NS
PW = E // NW
CH = 128
NCHUNK = PW // CH
RPT = N // NS

_f32 = jnp.float32
_i32 = jnp.int32


def _mesh():
    return plsc.VectorSubcoreMesh(core_axis_name="c", subcore_axis_name="s")


def _sc_gather_scalar(nodes1d, rows128, cols128):
    @functools.partial(
        pl.kernel, mesh=_mesh(),
        out_type=(jax.ShapeDtypeStruct((E,), _f32),
                  jax.ShapeDtypeStruct((E,), _f32)),
        scratch_types=[
            pltpu.VMEM((NCHUNK, CH), _i32), pltpu.VMEM((NCHUNK, CH), _i32),
            pltpu.VMEM((PW,), _f32), pltpu.VMEM((PW,), _f32),
            pltpu.SemaphoreType.DMA, pltpu.SemaphoreType.DMA,
        ],
    )
    def k(nodes_hbm, rows_hbm, cols_hbm, gr_hbm, gc_hbm, ri, ci, gr, gc, sa, sb):
        wid = lax.axis_index("s") * NC + lax.axis_index("c")
        base = wid * PW
        pltpu.sync_copy(rows_hbm.at[pl.ds(wid * NCHUNK, NCHUNK)], ri)
        pltpu.sync_copy(cols_hbm.at[pl.ds(wid * NCHUNK, NCHUNK)], ci)

        def fire(kk, carry):
            o = kk * CH
            pltpu.async_copy(nodes_hbm.at[ri.at[kk]], gr.at[pl.ds(o, CH)], sa)
            pltpu.async_copy(nodes_hbm.at[ci.at[kk]], gc.at[pl.ds(o, CH)], sb)
            return carry

        lax.fori_loop(0, NCHUNK, fire, 0)

        def drain(kk, carry):
            o = kk * CH
            pltpu.make_async_copy(nodes_hbm.at[ri.at[kk]],
                                  gr.at[pl.ds(o, CH)], sa).wait()
            pltpu.make_async_copy(nodes_hbm.at[ci.at[kk]],
                                  gc.at[pl.ds(o, CH)], sb).wait()
            return carry

        lax.fori_loop(0, NCHUNK, drain, 0)
        pltpu.sync_copy(gr, gr_hbm.at[pl.ds(base, PW)])
        pltpu.sync_copy(gc, gc_hbm.at[pl.ds(base, PW)])

    return k(nodes1d, rows128, cols128)


CHP = 64
NCHP = PW // CHP


def _sc_gather_pairs(A, B, rows64, cols64):
    @functools.partial(
        pl.kernel, mesh=_mesh(),
        out_type=jax.ShapeDtypeStruct((E, H), _f32),
        scratch_types=[
            pltpu.VMEM((NCHP, CHP), _i32), pltpu.VMEM((NCHP, CHP), _i32),
            pltpu.VMEM((CHP, H), _f32), pltpu.VMEM((CHP, H), _f32),
            pltpu.VMEM((CHP, H), _f32), pltpu.VMEM((CHP, H), _f32),
            pltpu.VMEM((CHP, H), _f32), pltpu.VMEM((CHP, H), _f32),
            pltpu.SemaphoreType.DMA, pltpu.SemaphoreType.DMA,
            pltpu.SemaphoreType.DMA, pltpu.SemaphoreType.DMA,
            pltpu.SemaphoreType.DMA, pltpu.SemaphoreType.DMA,
        ],
    )
    def k(a_hbm, b_hbm, rows_hbm, cols_hbm, g_hbm,
          ri, ci, av0, av1, bv0, bv1, ov0, ov1,
          sga0, sga1, sgb0, sgb1, sw0, sw1):
        av = (av0, av1)
        bv = (bv0, bv1)
        ov = (ov0, ov1)
        sga = (sga0, sga1)
        sgb = (sgb0, sgb1)
        sw = (sw0, sw1)
        wid = lax.axis_index("s") * NC + lax.axis_index("c")
        base = wid * PW
        pltpu.sync_copy(rows_hbm.at[pl.ds(wid * NCHP, NCHP)], ri)
        pltpu.sync_copy(cols_hbm.at[pl.ds(wid * NCHP, NCHP)], ci)
        for b in range(2):
            pltpu.async_copy(a_hbm.at[ri.at[b]], av[b], sga[b])
            pltpu.async_copy(b_hbm.at[ci.at[b]], bv[b], sgb[b])

        def k2body(k2, carry):
            for b in range(2):
                j = k2 * 2 + b
                pltpu.make_async_copy(a_hbm.at[ri.at[j]], av[b], sga[b]).wait()
                pltpu.make_async_copy(b_hbm.at[ci.at[j]], bv[b], sgb[b]).wait()

                @pl.when(j >= 2)
                def _():
                    pltpu.make_async_copy(
                        ov[b], g_hbm.at[pl.ds(base + (j - 2) * CHP, CHP)],
                        sw[b]).wait()

                def addrow(e, c2):
                    for t in range(H // 16):
                        sl = pl.ds(t * 16, 16)
                        ov[b][e, sl] = av[b][e, sl] + bv[b][e, sl]
                    return c2

                lax.fori_loop(0, CHP, addrow, 0)

                @pl.when(j + 2 < NCHP)
                def _():
                    pltpu.async_copy(a_hbm.at[ri.at[j + 2]], av[b], sga[b])
                    pltpu.async_copy(b_hbm.at[ci.at[j + 2]], bv[b], sgb[b])

                pltpu.async_copy(ov[b], g_hbm.at[pl.ds(base + j * CHP, CHP)],
                                 sw[b])
            return carry

        lax.fori_loop(0, NCHP // 2, k2body, 0)
        for b in range(2):
            j = NCHP - 2 + b
            pltpu.make_async_copy(
                ov[b], g_hbm.at[pl.ds(base + j * CHP, CHP)], sw[b]).wait()

    return k(A, B, rows64, cols64)


HH = H // NC
PS = E // NS
NCHUNK2 = PS // CH


def _sc_scatter_add(m2, rows):
    @functools.partial(
        pl.kernel, mesh=_mesh(),
        out_type=jax.ShapeDtypeStruct((N, H), _f32),
        scratch_types=[
            pltpu.VMEM((NCHUNK2, CH), _i32),
            pltpu.VMEM((CH, HH), _f32),
            pltpu.VMEM((CH, HH), _f32),
            pltpu.VMEM((CH, HH), _f32),
            pltpu.VMEM_SHARED((N, HH), _f32),
            pltpu.SemaphoreType.DMA, pltpu.SemaphoreType.DMA,
            pltpu.SemaphoreType.DMA, pltpu.SemaphoreType.DMA,
        ],
    )
    def k(m_hbm, rows_hbm, agg_hbm, riv, mv0, mv1, zv, acc,
          sr0, sr1, ss0, ss1):
        mv = (mv0, mv1)
        sr = (sr0, sr1)
        ss = (ss0, ss1)
        cid = lax.axis_index("c")
        sid = lax.axis_index("s")
        base = sid * PS
        fcol = cid * HH
        pltpu.sync_copy(rows_hbm.at[pl.ds(sid * NCHUNK2, NCHUNK2)], riv)

        def zbody(e, carry):
            for t in range(HH // 16):
                zv[e, pl.ds(t * 16, 16)] = jnp.zeros((16,), _f32)
            return carry

        lax.fori_loop(0, CH, zbody, 0)
        pltpu.sync_copy(zv, acc.at[pl.ds(sid * RPT, CH)])
        pltpu.sync_copy(zv, acc.at[pl.ds(sid * RPT + CH, CH)])
        plsc.subcore_barrier()

        for b in range(2):
            pltpu.async_copy(
                m_hbm.at[pl.ds(base + b * CH, CH), pl.ds(fcol, HH)],
                mv[b], sr[b])

        def k2body(k2, carry):
            for b in range(2):
                j = k2 * 2 + b
                pltpu.make_async_copy(
                    m_hbm.at[pl.ds(base + j * CH, CH), pl.ds(fcol, HH)],
                    mv[b], sr[b]).wait()
                pltpu.async_copy(mv[b], acc.at[riv.at[j]], ss[b], add=True)

                @pl.when(j + 2 < NCHUNK2)
                def _():
                    pltpu.make_async_copy(mv[b], acc.at[riv.at[j]],
                                          ss[b]).wait()
                    pltpu.async_copy(
                        m_hbm.at[pl.ds(base + (j + 2) * CH, CH),
                                 pl.ds(fcol, HH)],
                        mv[b], sr[b])
            return carry

        lax.fori_loop(0, NCHUNK2 // 2, k2body, 0)
        for b in range(2):
            pltpu.make_async_copy(
                mv[b], acc.at[riv.at[NCHUNK2 - 2 + b]], ss[b]).wait()
        plsc.subcore_barrier()
        pltpu.sync_copy(acc.at[pl.ds(sid * RPT, RPT)],
                        agg_hbm.at[pl.ds(sid * RPT, RPT), pl.ds(fcol, HH)])

    return k(m2, rows)


_BE = 2048


def _tc_edge0(gr, gc, ea, w_r, w_c, w_e, b1, w2, b2):
    def body(gr_ref, gc_ref, ea_ref, wr, wc, we, b1r, w2r, b2r, out):
        z = (gr_ref[...] * wr[...] + gc_ref[...] * wc[...]
             + ea_ref[...] * we[...] + b1r[...])
        m = jnp.maximum(z, 0.0)
        mm = jnp.dot(m, w2r[...], preferred_element_type=_f32) + b2r[...]
        out[...] = jnp.maximum(mm, 0.0)

    col = pl.BlockSpec((_BE, 1), lambda i: (i, 0))
    whole = lambda shape: pl.BlockSpec(shape, lambda i: (0,) * len(shape))
    return pl.pallas_call(
        body,
        grid=(E // _BE,),
        in_specs=[col, col, col, whole((1, H)), whole((1, H)), whole((1, H)),
                  whole((1, H)), whole((H, H)), whole((1, H))],
        out_specs=pl.BlockSpec((_BE, H), lambda i: (i, 0)),
        out_shape=jax.ShapeDtypeStruct((E, H), _f32),
    )(gr, gc, ea, w_r, w_c, w_e, b1, w2, b2)


def _tc_edge1(G, ea, w_e, w2, b2):
    def body(g_ref, ea_ref, we, w2r, b2r, out):
        z = g_ref[...] + ea_ref[...] * we[...]
        m = jnp.maximum(z, 0.0)
        mm = jnp.dot(m, w2r[...], preferred_element_type=_f32) + b2r[...]
        out[...] = jnp.maximum(mm, 0.0)

    whole = lambda shape: pl.BlockSpec(shape, lambda i: (0,) * len(shape))
    return pl.pallas_call(
        body,
        grid=(E // _BE,),
        in_specs=[pl.BlockSpec((_BE, H), lambda i: (i, 0)),
                  pl.BlockSpec((_BE, 1), lambda i: (i, 0)),
                  whole((1, H)), whole((H, H)), whole((1, H))],
        out_specs=pl.BlockSpec((_BE, H), lambda i: (i, 0)),
        out_shape=jax.ShapeDtypeStruct((E, H), _f32),
    )(G, ea, w_e, w2, b2)


_BN = 512


def _tc_node0(agg, nodes, w_h, w_a, b1, w2, b2, wA, bA, wB):
    def body(aggr, nd, whr, war, b1r, w2r, b2r, wAr, bAr, wBr, h1o, ao, bo):
        t = (nd[...] * whr[...]
             + jnp.dot(aggr[...], war[...], preferred_element_type=_f32) + b1r[...])
        u = jnp.maximum(t, 0.0)
        h = jnp.dot(u, w2r[...], preferred_element_type=_f32) + b2r[...] + nd[...]
        h1o[...] = h
        ao[...] = jnp.dot(h, wAr[...], preferred_element_type=_f32) + bAr[...]
        bo[...] = jnp.dot(h, wBr[...], preferred_element_type=_f32)

    whole = lambda shape: pl.BlockSpec(shape, lambda i: (0,) * len(shape))
    blk = pl.BlockSpec((_BN, H), lambda i: (i, 0))
    return pl.pallas_call(
        body,
        grid=(N // _BN,),
        in_specs=[blk,
                  pl.BlockSpec((_BN, 1), lambda i: (i, 0)),
                  whole((1, H)), whole((H, H)), whole((1, H)),
                  whole((H, H)), whole((1, H)),
                  whole((H, H)), whole((1, H)), whole((H, H))],
        out_specs=(blk, blk, blk),
        out_shape=(jax.ShapeDtypeStruct((N, H), _f32),
                   jax.ShapeDtypeStruct((N, H), _f32),
                   jax.ShapeDtypeStruct((N, H), _f32)),
    )(agg, nodes, w_h, w_a, b1, w2, b2, wA, bA, wB)


def _tc_node1(agg, h1, w_h, w_a, b1, w2, b2, ew, eb):
    def body(aggr, h1r, whr, war, b1r, w2r, b2r, ewr, ebr, xo):
        t = (jnp.dot(h1r[...], whr[...], preferred_element_type=_f32)
             + jnp.dot(aggr[...], war[...], preferred_element_type=_f32) + b1r[...])
        u = jnp.maximum(t, 0.0)
        h = (jnp.dot(u, w2r[...], preferred_element_type=_f32) + b2r[...]
             + h1r[...])
        xo[...] = jnp.dot(h, ewr[...], preferred_element_type=_f32) + ebr[...]

    whole = lambda shape: pl.BlockSpec(shape, lambda i: (0,) * len(shape))
    blk = pl.BlockSpec((_BN, H), lambda i: (i, 0))
    return pl.pallas_call(
        body,
        grid=(N // _BN,),
        in_specs=[blk, blk,
                  whole((H, H)), whole((H, H)), whole((1, H)),
                  whole((H, H)), whole((1, H)),
                  whole((H, EMB)), whole((1, EMB))],
        out_specs=pl.BlockSpec((_BN, EMB), lambda i: (i, 0)),
        out_shape=jax.ShapeDtypeStruct((N, EMB), _f32),
    )(agg, h1, w_h, w_a, b1, w2, b2, ew, eb)


_BR = 256


def _tc_decode(x, xT):
    def body(x_ref, xt_ref, out):
        xb = x_ref[...]
        xt = xt_ref[...]
        rb = jnp.sum(xb * xb, axis=1, keepdims=True)
        ra = jnp.sum(xt * xt, axis=0, keepdims=True)
        s = jnp.dot(xb, xt, preferred_element_type=_f32)
        t = 10.0 * (rb + ra - 2.0 * s) - 1.0
        sig = 1.0 / (1.0 + jnp.exp(-t))
        i = pl.program_id(0)
        rowid = lax.broadcasted_iota(_i32, (_BR, N), 0) + i * _BR
        colid = lax.broadcasted_iota(_i32, (_BR, N), 1)
        out[...] = jnp.where(rowid == colid, 0.0, sig)

    return pl.pallas_call(
        body,
        grid=(N // _BR,),
        in_specs=[pl.BlockSpec((_BR, EMB), lambda i: (i, 0)),
                  pl.BlockSpec((EMB, N), lambda i: (0, 0))],
        out_specs=pl.BlockSpec((_BR, N), lambda i: (i, 0)),
        out_shape=jax.ShapeDtypeStruct((N, N), _f32),
    )(x, xT)


def kernel(nodes, edges, edge_attr,
           g0_ew1, g0_eb1, g0_ew2, g0_eb2, g0_nw1, g0_nb1, g0_nw2, g0_nb2,
           g1_ew1, g1_eb1, g1_ew2, g1_eb2, g1_nw1, g1_nb1, g1_nw2, g1_nb2,
           emb_w, emb_b):
    rows = edges[0]
    cols = edges[1]
    rows128 = rows.reshape(E // CH, CH)
    cols128 = cols.reshape(E // CH, CH)
    rows64 = rows.reshape(E // CHP, CHP)
    cols64 = cols.reshape(E // CHP, CHP)

    gr, gc = _sc_gather_scalar(nodes.reshape(N), rows128, cols128)
    m2 = _tc_edge0(gr.reshape(E, 1), gc.reshape(E, 1), edge_attr,
                   g0_ew1[0:1], g0_ew1[1:2], g0_ew1[2:3],
                   g0_eb1.reshape(1, H), g0_ew2, g0_eb2.reshape(1, H))
    agg0 = _sc_scatter_add(m2, rows128)
    h1, A1, B1 = _tc_node0(agg0, nodes,
                           g0_nw1[0:1], g0_nw1[1:], g0_nb1.reshape(1, H),
                           g0_nw2, g0_nb2.reshape(1, H),
                           g1_ew1[0:H], g1_eb1.reshape(1, H), g1_ew1[H:2 * H])

    G = _sc_gather_pairs(A1, B1, rows64, cols64)
    m2b = _tc_edge1(G, edge_attr, g1_ew1[2 * H:2 * H + 1],
                    g1_ew2, g1_eb2.reshape(1, H))
    agg1 = _sc_scatter_add(m2b, rows128)
    x = _tc_node1(agg1, h1,
                  g1_nw1[0:H], g1_nw1[H:], g1_nb1.reshape(1, H),
                  g1_nw2, g1_nb2.reshape(1, H),
                  emb_w, emb_b.reshape(1, EMB))

    adj = _tc_decode(x, x.T)
    return adj, x
